# baseline (speedup 1.0000x reference)
import jax
import jax.numpy as jnp
from jax import lax
from jax.experimental import pallas as pl
from jax.experimental.pallas import tpu as pltpu

KREP = 16
BR = 199 * KREP


def _body(thrA_ref, thrB_ref, embs_ref, pat_ref, out_ref):
    br, w = embs_ref.shape
    col = lax.broadcasted_iota(jnp.int32, (br, w), 1)
    mask = (col < thrA_ref[...]) | (col >= thrB_ref[...])
    out_ref[...] = embs_ref[...] + jnp.where(mask, pat_ref[...], 0.0)


def kernel(embs, seq_lengths, table):
    B, L, D = embs.shape
    W = 2 * D
    NR = B * L // 2
    embs2 = embs.reshape(NR, W)

    tblf = table[1:L + 1].reshape(L * D)
    pat = jnp.tile(jnp.concatenate([tblf, tblf]).reshape(L, W), (KREP, 1))

    sl = seq_lengths.astype(jnp.int32)
    h = jnp.arange(2 * NR, dtype=jnp.int32)
    badd = (h % L) < sl[h // L]
    addL = badd[0::2]
    addR = badd[1::2]
    thrA = jnp.where(addL, jnp.where(addR, W, D), 0).astype(jnp.int32)
    thrB = jnp.where(addR, jnp.where(addL, W, D), W).astype(jnp.int32)

    grid = (NR // BR,)
    out = pl.pallas_call(
        _body,
        grid=grid,
        in_specs=[
            pl.BlockSpec((BR, 1), lambda i: (i, 0)),
            pl.BlockSpec((BR, 1), lambda i: (i, 0)),
            pl.BlockSpec((BR, W), lambda i: (i, 0)),
            pl.BlockSpec((BR, W), lambda i: (0, 0)),
        ],
        out_specs=pl.BlockSpec((BR, W), lambda i: (i, 0)),
        out_shape=jax.ShapeDtypeStruct((NR, W), jnp.float32),
    )(thrA.reshape(NR, 1), thrB.reshape(NR, 1), embs2, pat)
    return out.reshape(B, L, D)


# speedup vs baseline: 10.0583x; 10.0583x over previous
import jax
import jax.numpy as jnp
from jax import lax
from jax.experimental import pallas as pl
from jax.experimental.pallas import tpu as pltpu

BB = 64


def _body(thrA_ref, thrB_ref, embs_ref, tbl_ref, out_ref):
    bb, w = embs_ref.shape
    ld = w // 2
    col = lax.broadcasted_iota(jnp.int32, (bb, w), 1)
    mask = (col < thrA_ref[...]) | ((col >= ld) & (col < thrB_ref[...]))
    out_ref[...] = embs_ref[...] + jnp.where(mask, tbl_ref[...], 0.0)


def kernel(embs, seq_lengths, table):
    B, L, D = embs.shape
    LD = L * D
    W = 2 * LD
    NR = B // 2
    embs2 = embs.reshape(NR, W)

    tblf = table[1:L + 1].reshape(LD)
    tbl2 = jnp.concatenate([tblf, tblf]).reshape(1, W)

    sl = seq_lengths.astype(jnp.int32) * D
    thrA = sl[0::2].reshape(NR, 1)
    thrB = (sl[1::2] + LD).reshape(NR, 1)

    grid = (NR // BB,)
    out = pl.pallas_call(
        _body,
        grid=grid,
        in_specs=[
            pl.BlockSpec((BB, 1), lambda i: (i, 0)),
            pl.BlockSpec((BB, 1), lambda i: (i, 0)),
            pl.BlockSpec((BB, W), lambda i: (i, 0)),
            pl.BlockSpec((1, W), lambda i: (0, 0)),
        ],
        out_specs=pl.BlockSpec((BB, W), lambda i: (i, 0)),
        out_shape=jax.ShapeDtypeStruct((NR, W), jnp.float32),
    )(thrA, thrB, embs2, tbl2)
    return out.reshape(B, L, D)


# speedup vs baseline: 21.9409x; 2.1814x over previous
import jax
import jax.numpy as jnp
from jax import lax
from jax.experimental import pallas as pl
from jax.experimental.pallas import tpu as pltpu


def _body(thresh_ref, embs_ref, tbl_ref, out_ref):
    bb, ld = embs_ref.shape
    col = lax.broadcasted_iota(jnp.int32, (bb, ld), 1)
    mask = col < thresh_ref[...]
    out_ref[...] = embs_ref[...] + jnp.where(mask, tbl_ref[...], 0.0)


def kernel(embs, seq_lengths, table):
    B, L, D = embs.shape
    LD = L * D
    embs2 = embs.reshape(B, LD)
    tbl = table[1:L + 1].reshape(1, LD)
    thresh = (seq_lengths.astype(jnp.int32) * D).reshape(B, 1)

    BB = 128
    grid = (B // BB,)
    out = pl.pallas_call(
        _body,
        grid=grid,
        in_specs=[
            pl.BlockSpec((BB, 1), lambda i: (i, 0)),
            pl.BlockSpec((BB, LD), lambda i: (i, 0)),
            pl.BlockSpec((1, LD), lambda i: (0, 0)),
        ],
        out_specs=pl.BlockSpec((BB, LD), lambda i: (i, 0)),
        out_shape=jax.ShapeDtypeStruct((B, LD), jnp.float32),
    )(thresh, embs2, tbl)
    return out.reshape(B, L, D)
